# Initial kernel scaffold; baseline (speedup 1.0000x reference)
import jax
import jax.numpy as jnp
from jax.experimental import pallas as pl


def kernel(indices, embed):
    raise NotImplementedError("write your pallas kernel here")


# speedup vs baseline: 5.1774x; 5.1774x over previous
import functools

import jax
import jax.numpy as jnp
from jax import lax
from jax.experimental import pallas as pl
from jax.experimental.pallas import tpu as pltpu
from jax.experimental.pallas import tpu_sc as plsc

DIM = 128
CHUNK = 128


@functools.lru_cache(maxsize=None)
def _make_gather(B: int):
    info = plsc.get_sparse_core_info()
    nc, ns = info.num_cores, info.num_subcores
    nw = nc * ns
    assert B % (nw * CHUNK) == 0
    b_per_w = B // nw
    n_chunks = b_per_w // CHUNK
    mesh = plsc.VectorSubcoreMesh(core_axis_name="c", subcore_axis_name="s")

    @functools.partial(
        pl.kernel,
        out_type=jax.ShapeDtypeStruct((B, DIM), jnp.float32),
        mesh=mesh,
        scratch_types=[
            pltpu.VMEM((CHUNK,), jnp.int32),
            pltpu.VMEM((CHUNK, DIM), jnp.float32),
            pltpu.SemaphoreType.DMA,
        ],
    )
    def gather(idx_hbm, table_hbm, out_hbm, idx_v, rows_v, sem):
        wid = lax.axis_index("s") * nc + lax.axis_index("c")
        wbase = wid * b_per_w

        @pl.loop(0, n_chunks)
        def _(j):
            base = wbase + j * CHUNK
            pltpu.sync_copy(idx_hbm.at[pl.ds(base, CHUNK)], idx_v)
            pltpu.async_copy(table_hbm.at[idx_v], rows_v, sem).wait()
            pltpu.sync_copy(rows_v, out_hbm.at[pl.ds(base, CHUNK)])

    return gather


@jax.jit
def kernel(indices, embed):
    bsz, hist = indices.shape
    flat = indices.reshape(bsz * hist)
    out = _make_gather(bsz * hist)(flat, embed)
    return out.reshape(bsz, hist, DIM)


# speedup vs baseline: 9.2341x; 1.7835x over previous
import functools

import jax
import jax.numpy as jnp
from jax import lax
from jax.experimental import pallas as pl
from jax.experimental.pallas import tpu as pltpu
from jax.experimental.pallas import tpu_sc as plsc

DIM = 128
CHUNK = 128
NBUF = 4


@functools.lru_cache(maxsize=None)
def _make_gather(B: int):
    info = plsc.get_sparse_core_info()
    nc, ns = info.num_cores, info.num_subcores
    nw = nc * ns
    assert B % (nw * CHUNK * NBUF) == 0
    b_per_w = B // nw
    n_chunks = b_per_w // CHUNK
    n_groups = n_chunks // NBUF
    mesh = plsc.VectorSubcoreMesh(core_axis_name="c", subcore_axis_name="s")

    @functools.partial(
        pl.kernel,
        out_type=jax.ShapeDtypeStruct((B, DIM), jnp.float32),
        mesh=mesh,
        scratch_types=[
            pltpu.VMEM((NBUF, CHUNK), jnp.int32),
            pltpu.VMEM((NBUF, CHUNK, DIM), jnp.float32),
        ]
        + [pltpu.SemaphoreType.DMA] * (2 * NBUF),
    )
    def gather(idx_hbm, table_hbm, out_hbm, idx_v, rows_v, *sems):
        gsems, ssems = sems[:NBUF], sems[NBUF:]
        wid = lax.axis_index("s") * nc + lax.axis_index("c")
        wbase = wid * b_per_w

        def start_gather(i, b):
            pltpu.sync_copy(idx_hbm.at[pl.ds(wbase + i * CHUNK, CHUNK)],
                            idx_v.at[b])
            pltpu.async_copy(table_hbm.at[idx_v.at[b]], rows_v.at[b], gsems[b])

        def wait_gather(b):
            pltpu.make_async_copy(table_hbm.at[idx_v.at[b]], rows_v.at[b],
                                  gsems[b]).wait()

        def start_store(i, b):
            pltpu.async_copy(rows_v.at[b],
                             out_hbm.at[pl.ds(wbase + i * CHUNK, CHUNK)],
                             ssems[b])

        def wait_store(i, b):
            pltpu.make_async_copy(rows_v.at[b],
                                  out_hbm.at[pl.ds(wbase + i * CHUNK, CHUNK)],
                                  ssems[b]).wait()

        for b in range(NBUF):
            start_gather(b, b)

        @pl.loop(0, n_groups - 1)
        def _(g):
            for b in range(NBUF):
                wait_gather(b)
                start_store(g * NBUF + b, b)
            for b in range(NBUF):
                wait_store(g * NBUF + b, b)
                start_gather((g + 1) * NBUF + b, b)

        last = (n_groups - 1) * NBUF
        for b in range(NBUF):
            wait_gather(b)
            start_store(last + b, b)
        for b in range(NBUF):
            wait_store(last + b, b)

    return gather


@jax.jit
def kernel(indices, embed):
    bsz, hist = indices.shape
    flat = indices.reshape(bsz * hist)
    out = _make_gather(bsz * hist)(flat, embed)
    return out.reshape(bsz, hist, DIM)
